# Initial kernel scaffold; baseline (speedup 1.0000x reference)
import jax
import jax.numpy as jnp
from jax.experimental import pallas as pl


def kernel(x, edge_index, W, b):
    raise NotImplementedError("write your pallas kernel here")


# speedup vs baseline: 7.4992x; 7.4992x over previous
import functools

import jax
import jax.numpy as jnp
from jax import lax
from jax.experimental import pallas as pl
from jax.experimental.pallas import tpu as pltpu
from jax.experimental.pallas import tpu_sc as plsc

N = 10000
E = 320000
D = 128

NC = 2
NS = 16
NW = NC * NS

C = 80
EW = E // NW
NCH = EW // C

RPB = C
NB = N // RPB
KMAX = -(-NB // NS)


@functools.partial(
    pl.kernel,
    out_type=jax.ShapeDtypeStruct((NC, N, D), jnp.float32),
    mesh=plsc.VectorSubcoreMesh(
        core_axis_name="c", subcore_axis_name="s",
        num_cores=NC, num_subcores=NS),
    scratch_types=[
        pltpu.VMEM((NCH, C), jnp.int32),
        pltpu.VMEM((NCH, C), jnp.int32),
        pltpu.VMEM((C, D), jnp.float32),
        pltpu.VMEM_SHARED((N, D), jnp.float32),
        pltpu.SemaphoreType.DMA,
    ],
)
def _sc_aggregate(x_hbm, src_hbm, dst_hbm, out_hbm,
                  sall, dall, rows, agg, sem):
    cid = lax.axis_index("c")
    sid = lax.axis_index("s")
    wid = cid * NS + sid

    pltpu.sync_copy(src_hbm.at[wid], sall)
    pltpu.sync_copy(dst_hbm.at[wid], dall)

    def zrow(i, _):
        def zlane(j, _):
            rows[i, pl.ds(j * 16, 16)] = jnp.zeros((16,), jnp.float32)
            return 0
        return lax.fori_loop(0, D // 16, zlane, 0)
    lax.fori_loop(0, RPB, zrow, 0)

    def zcp(k, _):
        cb = sid + k * NS
        @pl.when(cb < NB)
        def _():
            pltpu.sync_copy(rows, agg.at[pl.ds(cb * RPB, RPB)])
        return 0
    lax.fori_loop(0, KMAX, zcp, 0)
    plsc.subcore_barrier()

    def chunk(j, _):
        pltpu.async_copy(x_hbm.at[sall.at[j]], rows, sem).wait()
        pltpu.sync_copy(rows, agg.at[dall.at[j]], add=True)
        return 0
    lax.fori_loop(0, NCH, chunk, 0)
    plsc.subcore_barrier()

    def ocp(k, _):
        cb = sid + k * NS
        @pl.when(cb < NB)
        def _():
            r0 = cb * RPB
            pltpu.sync_copy(agg.at[pl.ds(r0, RPB)], rows)
            pltpu.sync_copy(rows, out_hbm.at[cid, pl.ds(r0, RPB)])
        return 0
    lax.fori_loop(0, KMAX, ocp, 0)


BR = 1000


def _tc_update_body(x_ref, p0_ref, p1_ref, w_ref, b_ref, o_ref):
    h = x_ref[...] + p0_ref[...] + p1_ref[...]
    acc = jnp.dot(h, w_ref[...], preferred_element_type=jnp.float32)
    o_ref[...] = jnp.maximum(acc + b_ref[...], 0.0)


def _tc_update(x, p0, p1, W, b2):
    return pl.pallas_call(
        _tc_update_body,
        grid=(N // BR,),
        in_specs=[
            pl.BlockSpec((BR, D), lambda i: (i, 0)),
            pl.BlockSpec((BR, D), lambda i: (i, 0)),
            pl.BlockSpec((BR, D), lambda i: (i, 0)),
            pl.BlockSpec((D, D), lambda i: (0, 0)),
            pl.BlockSpec((1, D), lambda i: (0, 0)),
        ],
        out_specs=pl.BlockSpec((BR, D), lambda i: (i, 0)),
        out_shape=jax.ShapeDtypeStruct((N, D), jnp.float32),
    )(x, p0, p1, W, b2)


def kernel(x, edge_index, W, b):
    src = edge_index[0].astype(jnp.int32).reshape(NW, NCH, C)
    dst = edge_index[1].astype(jnp.int32).reshape(NW, NCH, C)
    parts = _sc_aggregate(x, src, dst)
    return _tc_update(x, parts[0], parts[1], W, b.reshape(1, D))
